# speedup vs baseline: 6.0680x; 6.0680x over previous
import functools

import jax
import jax.numpy as jnp
from jax import lax
from jax.experimental import pallas as pl
from jax.experimental.pallas import tpu as pltpu

N = 100000
D = 128
G = 512
BLK = 2048


def _body(x_ref, b_ref, wl_ref, bl_ref, wg_ref, bg_ref, wf_ref, bf_ref,
          out_ref, acc_ref, cnt_ref):
    i = pl.program_id(0)
    nb = pl.num_programs(0)

    @pl.when(i == 0)
    def _init():
        acc_ref[...] = jnp.zeros_like(acc_ref)
        cnt_ref[...] = jnp.zeros_like(cnt_ref)

    x = x_ref[...]
    s = lax.dot_general(x, wl_ref[...], (((1,), (1,)), ((), ())),
                        preferred_element_type=jnp.float32) + bl_ref[...]
    g = lax.dot_general(x, wg_ref[...], (((1,), (1,)), ((), ())),
                        preferred_element_type=jnp.float32) + bg_ref[...]
    g = g - jnp.max(g, axis=1, keepdims=True)
    g = jnp.exp(g)
    g = g / jnp.sum(g, axis=1, keepdims=True)
    h = s * g

    ids = b_ref[...].reshape(1, BLK)
    onehot = (lax.broadcasted_iota(jnp.int32, (G, BLK), 0) == ids
              ).astype(jnp.float32)
    acc_ref[...] += lax.dot_general(onehot, h, (((1,), (0,)), ((), ())),
                                    preferred_element_type=jnp.float32)
    cnt_ref[...] += jnp.sum(onehot, axis=1, keepdims=True)

    @pl.when(i == nb - 1)
    def _fin():
        mean = acc_ref[...] / jnp.maximum(cnt_ref[...], 1.0)
        out_ref[...] = lax.dot_general(
            mean, wf_ref[...], (((1,), (1,)), ((), ())),
            preferred_element_type=jnp.float32) + bf_ref[...]


@functools.partial(jax.jit, static_argnames=("interpret",))
def kernel(x, batch, W_lin, b_lin, W_gate, b_gate, W_final, b_final,
           interpret=False):
    n = x.shape[0]
    nb = (n + BLK - 1) // BLK
    n_pad = nb * BLK
    if n_pad != n:
        x = jnp.pad(x, ((0, n_pad - n), (0, 0)))
        batch = jnp.pad(batch, (0, n_pad - n), constant_values=G)
    batch3 = batch.reshape(nb, 1, BLK)

    wspec = pl.BlockSpec((D, D), lambda i: (0, 0))
    bspec = pl.BlockSpec((1, D), lambda i: (0, 0))
    out = pl.pallas_call(
        _body,
        grid=(nb,),
        in_specs=[
            pl.BlockSpec((BLK, D), lambda i: (i, 0)),
            pl.BlockSpec((1, 1, BLK), lambda i: (i, 0, 0)),
            wspec, bspec, wspec, bspec, wspec, bspec,
        ],
        out_specs=pl.BlockSpec((G, D), lambda i: (0, 0)),
        out_shape=jax.ShapeDtypeStruct((G, D), jnp.float32),
        scratch_shapes=[
            pltpu.VMEM((G, D), jnp.float32),
            pltpu.VMEM((G, 1), jnp.float32),
        ],
        compiler_params=pltpu.CompilerParams(
            dimension_semantics=("arbitrary",)),
        interpret=interpret,
    )(x, batch3, W_lin, b_lin.reshape(1, D), W_gate, b_gate.reshape(1, D),
      W_final, b_final.reshape(1, D))
    return out
